# baseline (speedup 1.0000x reference)
import functools

import jax
import jax.numpy as jnp
from jax import lax
from jax.experimental import pallas as pl
from jax.experimental.pallas import tpu as pltpu
from jax.experimental.pallas import tpu_sc as plsc

LANE = 128
NC = 2
NS = 16
NW = NC * NS
VREG = 16


def _sc_degree(dst3, ew3, n_pad):
    nw, nch, lane = dst3.shape
    rpt = n_pad // NS
    mesh = plsc.VectorSubcoreMesh(core_axis_name="c", subcore_axis_name="s")

    @functools.partial(
        pl.kernel,
        out_type=jax.ShapeDtypeStruct((NC, n_pad), jnp.float32),
        mesh=mesh,
        scratch_types=[
            pltpu.VMEM((nch, lane), jnp.int32),
            pltpu.VMEM((nch, lane), jnp.float32),
            pltpu.VMEM((rpt,), jnp.float32),
            pltpu.VMEM_SHARED((n_pad,), jnp.float32),
        ],
        compiler_params=pltpu.CompilerParams(needs_layout_passes=False, use_tc_tiling_on_sc=False),
    )
    def deg_kernel(dst_hbm, ew_hbm, out_hbm, dst_v, ew_v, buf_v, acc_sh):
        c = lax.axis_index("c")
        s = lax.axis_index("s")
        w = s * NC + c
        pltpu.sync_copy(dst_hbm.at[w], dst_v)
        pltpu.sync_copy(ew_hbm.at[w], ew_v)

        def zero_body(i, carry):
            buf_v[pl.ds(i * VREG, VREG)] = jnp.zeros((VREG,), jnp.float32)
            return carry

        lax.fori_loop(0, rpt // VREG, zero_body, 0)
        pltpu.sync_copy(buf_v, acc_sh.at[pl.ds(s * rpt, rpt)])
        plsc.subcore_barrier()

        def edge_body(j, carry):
            pltpu.sync_copy(ew_v.at[j], acc_sh.at[dst_v.at[j]], add=True)
            return carry

        lax.fori_loop(0, nch, edge_body, 0)
        plsc.subcore_barrier()

        pltpu.sync_copy(acc_sh.at[pl.ds(s * rpt, rpt)], buf_v)
        pltpu.sync_copy(buf_v, out_hbm.at[c, pl.ds(s * rpt, rpt)])

    return deg_kernel(dst3, ew3)


def _sc_edge_pass(y, dis, src3, dst3, ew3):
    n_pad, d = y.shape
    nw, nch, lane = src3.shape
    rpt = n_pad // NS
    mesh = plsc.VectorSubcoreMesh(core_axis_name="c", subcore_axis_name="s")

    @functools.partial(
        pl.kernel,
        out_type=jax.ShapeDtypeStruct((NC, n_pad, d), jnp.float32),
        mesh=mesh,
        scratch_types=[
            pltpu.VMEM((nch, lane), jnp.int32),
            pltpu.VMEM((nch, lane), jnp.int32),
            pltpu.VMEM((nch, lane), jnp.float32),
            pltpu.VMEM((n_pad,), jnp.float32),
            pltpu.VMEM((lane, d), jnp.float32),
            pltpu.VMEM((lane, d), jnp.float32),
            pltpu.VMEM_SHARED((n_pad, d), jnp.float32),
            pltpu.SemaphoreType.DMA,
            pltpu.SemaphoreType.DMA,
            pltpu.SemaphoreType.DMA,
            pltpu.SemaphoreType.DMA,
        ],
        compiler_params=pltpu.CompilerParams(needs_layout_passes=False, use_tc_tiling_on_sc=False),
    )
    def edge_kernel(y_hbm, dis_hbm, src_hbm, dst_hbm, ew_hbm, out_hbm,
                    src_v, dst_v, ew_v, dis_v, buf0_v, buf1_v, acc_sh,
                    gsem0, gsem1, ssem0, ssem1):
        c = lax.axis_index("c")
        s = lax.axis_index("s")
        w = s * NC + c
        bufs = (buf0_v, buf1_v)
        gsems = (gsem0, gsem1)
        ssems = (ssem0, ssem1)
        pltpu.sync_copy(src_hbm.at[w], src_v)
        pltpu.sync_copy(dst_hbm.at[w], dst_v)
        pltpu.sync_copy(ew_hbm.at[w], ew_v)
        pltpu.sync_copy(dis_hbm, dis_v)

        def zrow(i, carry):
            for v in range(d // VREG):
                buf0_v[i, pl.ds(v * VREG, VREG)] = jnp.zeros((VREG,), jnp.float32)
            return carry

        lax.fori_loop(0, lane, zrow, 0)
        for k in range(rpt // lane):
            pltpu.sync_copy(buf0_v, acc_sh.at[pl.ds(s * rpt + k * lane, lane)])
        plsc.subcore_barrier()

        def scale_rows(j, buf):
            def grp_body(g, carry2):
                sl = pl.ds(g * VREG, VREG)
                nsrc = plsc.load_gather(dis_v, [src_v[j, sl]])
                ndst = plsc.load_gather(dis_v, [dst_v[j, sl]])
                nv = nsrc * ew_v[j, sl] * ndst
                eid = lax.iota(jnp.int32, VREG) + g * VREG
                for f in range(d):
                    col = jnp.full((VREG,), f, jnp.int32)
                    vals = plsc.load_gather(buf, [eid, col])
                    plsc.store_scatter(buf, [eid, col], vals * nv)
                return carry2

            lax.fori_loop(0, lane // VREG, grp_body, 0)

        pltpu.async_copy(y_hbm.at[src_v.at[0]], buf0_v, gsem0)

        def chunk_body(j, carry):
            def run(bb):
                buf, gsem, ssem = bufs[bb], gsems[bb], ssems[bb]
                obuf, ogsem, ossem = bufs[1 - bb], gsems[1 - bb], ssems[1 - bb]

                @pl.when(j + 1 < nch)
                def _():
                    @pl.when(j >= 1)
                    def _():
                        pltpu.make_async_copy(
                            obuf, acc_sh.at[dst_v.at[j - 1]], ossem).wait()

                    pltpu.async_copy(y_hbm.at[src_v.at[j + 1]], obuf, ogsem)

                pltpu.make_async_copy(y_hbm.at[src_v.at[j]], buf, gsem).wait()
                scale_rows(j, buf)
                pltpu.async_copy(buf, acc_sh.at[dst_v.at[j]], ssem, add=True)

            @pl.when(j % 2 == 0)
            def _():
                run(0)

            @pl.when(j % 2 == 1)
            def _():
                run(1)

            return carry

        lax.fori_loop(0, nch, chunk_body, 0)
        b_last = (nch - 1) % 2
        pltpu.make_async_copy(
            bufs[b_last], acc_sh.at[dst_v.at[nch - 1]], ssems[b_last]).wait()
        if nch >= 2:
            pltpu.make_async_copy(
                bufs[1 - b_last], acc_sh.at[dst_v.at[nch - 2]],
                ssems[1 - b_last]).wait()
        plsc.subcore_barrier()

        for k in range(rpt // lane):
            r0 = s * rpt + k * lane
            pltpu.sync_copy(acc_sh.at[pl.ds(r0, lane)], buf0_v)
            pltpu.sync_copy(buf0_v, out_hbm.at[c, pl.ds(r0, lane)])

    return edge_kernel(y, dis, src3, dst3, ew3)


def _tc1_body(x_ref, w_ref, dp_ref, ya_ref, yb_ref, dis_ref):
    xw = jnp.dot(x_ref[...], w_ref[...], preferred_element_type=jnp.float32)
    half = xw.shape[1] // 2
    ya_ref[...] = xw[:, :half]
    yb_ref[...] = xw[:, half:]
    deg = jnp.sum(dp_ref[...], axis=0)
    dis_ref[...] = jnp.where(deg > 0, lax.rsqrt(deg), 0.0)


def _tc_matmul_dis(x_p, w1, deg_part, br):
    n_pad, d = x_p.shape
    h = w1.shape[1]
    nw = deg_part.shape[0]
    return pl.pallas_call(
        _tc1_body,
        grid=(n_pad // br,),
        in_specs=[
            pl.BlockSpec((br, d), lambda i: (i, 0)),
            pl.BlockSpec((d, h), lambda i: (0, 0)),
            pl.BlockSpec((nw, br), lambda i: (0, i)),
        ],
        out_specs=[
            pl.BlockSpec((br, h // 2), lambda i: (i, 0)),
            pl.BlockSpec((br, h // 2), lambda i: (i, 0)),
            pl.BlockSpec((br,), lambda i: (i,)),
        ],
        out_shape=[
            jax.ShapeDtypeStruct((n_pad, h // 2), jnp.float32),
            jax.ShapeDtypeStruct((n_pad, h // 2), jnp.float32),
            jax.ShapeDtypeStruct((n_pad,), jnp.float32),
        ],
    )(x_p, w1, deg_part)


def _tc2_body(aa_ref, ab_ref, b1_ref, w2_ref, y2_ref):
    hid = jnp.concatenate([aa_ref[0] + aa_ref[1], ab_ref[0] + ab_ref[1]],
                          axis=1)
    hid = jnp.maximum(hid + b1_ref[...], 0.0)
    y2_ref[...] = jnp.dot(hid, w2_ref[...], preferred_element_type=jnp.float32)


def _tc_hidden(acc1a, acc1b, b1, w2p, br):
    _, n_pad, hh = acc1a.shape
    op = w2p.shape[1]
    return pl.pallas_call(
        _tc2_body,
        grid=(n_pad // br,),
        in_specs=[
            pl.BlockSpec((2, br, hh), lambda i: (0, i, 0)),
            pl.BlockSpec((2, br, hh), lambda i: (0, i, 0)),
            pl.BlockSpec((2 * hh,), lambda i: (0,)),
            pl.BlockSpec((2 * hh, op), lambda i: (0, 0)),
        ],
        out_specs=pl.BlockSpec((br, op), lambda i: (i, 0)),
        out_shape=jax.ShapeDtypeStruct((n_pad, op), jnp.float32),
    )(acc1a, acc1b, b1, w2p)


def _tc3_body(a_ref, b2_ref, o_ref):
    o_ref[...] = a_ref[0] + a_ref[1] + b2_ref[...]


def _tc_final(acc2, b2p, br):
    _, n_pad, op = acc2.shape
    return pl.pallas_call(
        _tc3_body,
        grid=(n_pad // br,),
        in_specs=[
            pl.BlockSpec((2, br, op), lambda i: (0, i, 0)),
            pl.BlockSpec((op,), lambda i: (0,)),
        ],
        out_specs=pl.BlockSpec((br, op), lambda i: (i, 0)),
        out_shape=jax.ShapeDtypeStruct((n_pad, op), jnp.float32),
    )(acc2, b2p)


def kernel(x, edge_index, edge_weight, W1, b1, W2, b2):
    n, d = x.shape
    o = W2.shape[1]
    br = 1024
    n_pad = -(-n // br) * br
    o_p = -(-o // VREG) * VREG
    e = edge_weight.shape[0]
    e_tot = e + n_pad
    ept = -(-e_tot // (NW * LANE)) * LANE
    e_pad = ept * NW
    nch = ept // LANE

    src = edge_index[0].astype(jnp.int32)
    dst = edge_index[1].astype(jnp.int32)
    loop_idx = jnp.arange(n_pad, dtype=jnp.int32)
    zpad_i = jnp.zeros((e_pad - e_tot,), jnp.int32)
    src3 = jnp.concatenate([src, loop_idx, zpad_i]).reshape(NW, nch, LANE)
    dst3 = jnp.concatenate([dst, loop_idx, zpad_i]).reshape(NW, nch, LANE)
    ew3 = jnp.concatenate([
        edge_weight.astype(jnp.float32),
        jnp.ones((n_pad,), jnp.float32),
        jnp.zeros((e_pad - e_tot,), jnp.float32),
    ]).reshape(NW, nch, LANE)
    x_p = jnp.pad(x, ((0, n_pad - n), (0, 0)))
    w2p = jnp.pad(W2, ((0, 0), (0, o_p - o)))
    b2p = jnp.pad(b2, ((0, o_p - o),))

    deg_part = _sc_degree(dst3, ew3, n_pad)
    y1a, y1b, dis = _tc_matmul_dis(x_p, W1, deg_part, br)
    acc1a = _sc_edge_pass(y1a, dis, src3, dst3, ew3)
    acc1b = _sc_edge_pass(y1b, dis, src3, dst3, ew3)
    y2 = _tc_hidden(acc1a, acc1b, b1, w2p, br)
    acc2 = _sc_edge_pass(y2, dis, src3, dst3, ew3)
    outp = _tc_final(acc2, b2p, br)
    return outp[:n, :o]


# speedup vs baseline: 4.5845x; 4.5845x over previous
import functools

import jax
import jax.numpy as jnp
from jax import lax
from jax.experimental import pallas as pl
from jax.experimental.pallas import tpu as pltpu
from jax.experimental.pallas import tpu_sc as plsc

LANE = 128
NC = 2
NS = 16
NW = NC * NS
VREG = 16


def _sc_degree(dst3, ew3, n_pad):
    nw, nch, lane = dst3.shape
    rpt = n_pad // NS
    mesh = plsc.VectorSubcoreMesh(core_axis_name="c", subcore_axis_name="s")

    @functools.partial(
        pl.kernel,
        out_type=jax.ShapeDtypeStruct((NC, n_pad), jnp.float32),
        mesh=mesh,
        scratch_types=[
            pltpu.VMEM((nch, lane), jnp.int32),
            pltpu.VMEM((nch, lane), jnp.float32),
            pltpu.VMEM((rpt,), jnp.float32),
            pltpu.VMEM_SHARED((n_pad,), jnp.float32),
        ],
        compiler_params=pltpu.CompilerParams(needs_layout_passes=False, use_tc_tiling_on_sc=False),
    )
    def deg_kernel(dst_hbm, ew_hbm, out_hbm, dst_v, ew_v, buf_v, acc_sh):
        c = lax.axis_index("c")
        s = lax.axis_index("s")
        w = s * NC + c
        pltpu.sync_copy(dst_hbm.at[w], dst_v)
        pltpu.sync_copy(ew_hbm.at[w], ew_v)

        def zero_body(i, carry):
            buf_v[pl.ds(i * VREG, VREG)] = jnp.zeros((VREG,), jnp.float32)
            return carry

        lax.fori_loop(0, rpt // VREG, zero_body, 0)
        pltpu.sync_copy(buf_v, acc_sh.at[pl.ds(s * rpt, rpt)])
        plsc.subcore_barrier()

        def edge_body(j, carry):
            pltpu.sync_copy(ew_v.at[j], acc_sh.at[dst_v.at[j]], add=True)
            return carry

        lax.fori_loop(0, nch, edge_body, 0)
        plsc.subcore_barrier()

        pltpu.sync_copy(acc_sh.at[pl.ds(s * rpt, rpt)], buf_v)
        pltpu.sync_copy(buf_v, out_hbm.at[c, pl.ds(s * rpt, rpt)])

    return deg_kernel(dst3, ew3)


def _sc_edge_pass(y, dis, src3, dst3, ew3):
    n_pad, d = y.shape
    nw, nch, lane = src3.shape
    rpt = n_pad // NS
    mesh = plsc.VectorSubcoreMesh(core_axis_name="c", subcore_axis_name="s")

    @functools.partial(
        pl.kernel,
        out_type=jax.ShapeDtypeStruct((NC, n_pad, d), jnp.float32),
        mesh=mesh,
        scratch_types=[
            pltpu.VMEM((nch, lane), jnp.int32),
            pltpu.VMEM((nch, lane), jnp.int32),
            pltpu.VMEM((nch, lane), jnp.float32),
            pltpu.VMEM((n_pad,), jnp.float32),
            pltpu.VMEM((lane, d), jnp.float32),
            pltpu.VMEM((lane, d), jnp.float32),
            pltpu.VMEM_SHARED((n_pad, d), jnp.float32),
            pltpu.SemaphoreType.DMA,
            pltpu.SemaphoreType.DMA,
            pltpu.SemaphoreType.DMA,
            pltpu.SemaphoreType.DMA,
        ],
        compiler_params=pltpu.CompilerParams(needs_layout_passes=False, use_tc_tiling_on_sc=False),
    )
    def edge_kernel(y_hbm, dis_hbm, src_hbm, dst_hbm, ew_hbm, out_hbm,
                    src_v, dst_v, ew_v, dis_v, buf0_v, buf1_v, acc_sh,
                    gsem0, gsem1, ssem0, ssem1):
        c = lax.axis_index("c")
        s = lax.axis_index("s")
        w = s * NC + c
        bufs = (buf0_v, buf1_v)
        gsems = (gsem0, gsem1)
        ssems = (ssem0, ssem1)
        pltpu.sync_copy(src_hbm.at[w], src_v)
        pltpu.sync_copy(dst_hbm.at[w], dst_v)
        pltpu.sync_copy(ew_hbm.at[w], ew_v)
        pltpu.sync_copy(dis_hbm, dis_v)

        def zrow(i, carry):
            for v in range(d // VREG):
                buf0_v[i, pl.ds(v * VREG, VREG)] = jnp.zeros((VREG,), jnp.float32)
            return carry

        lax.fori_loop(0, lane, zrow, 0)
        for k in range(rpt // lane):
            pltpu.sync_copy(buf0_v, acc_sh.at[pl.ds(s * rpt + k * lane, lane)])
        plsc.subcore_barrier()

        def scale_rows(j, buf):
            for g in range(lane // VREG):
                sl = pl.ds(g * VREG, VREG)
                nsrc = plsc.load_gather(dis_v, [src_v[j, sl]])
                ndst = plsc.load_gather(dis_v, [dst_v[j, sl]])
                nv = nsrc * ew_v[j, sl] * ndst
                for l in range(VREG):
                    e = g * VREG + l
                    scale = nv[l]
                    for v in range(d // VREG):
                        sl2 = pl.ds(v * VREG, VREG)
                        buf[e, sl2] = buf[e, sl2] * scale

        pltpu.async_copy(y_hbm.at[src_v.at[0]], buf0_v, gsem0)

        def chunk_body(j, carry):
            def run(bb):
                buf, gsem, ssem = bufs[bb], gsems[bb], ssems[bb]
                obuf, ogsem, ossem = bufs[1 - bb], gsems[1 - bb], ssems[1 - bb]

                @pl.when(j + 1 < nch)
                def _():
                    @pl.when(j >= 1)
                    def _():
                        pltpu.make_async_copy(
                            obuf, acc_sh.at[dst_v.at[j - 1]], ossem).wait()

                    pltpu.async_copy(y_hbm.at[src_v.at[j + 1]], obuf, ogsem)

                pltpu.make_async_copy(y_hbm.at[src_v.at[j]], buf, gsem).wait()
                scale_rows(j, buf)
                pltpu.async_copy(buf, acc_sh.at[dst_v.at[j]], ssem, add=True)

            @pl.when(j % 2 == 0)
            def _():
                run(0)

            @pl.when(j % 2 == 1)
            def _():
                run(1)

            return carry

        lax.fori_loop(0, nch, chunk_body, 0)
        b_last = (nch - 1) % 2
        pltpu.make_async_copy(
            bufs[b_last], acc_sh.at[dst_v.at[nch - 1]], ssems[b_last]).wait()
        if nch >= 2:
            pltpu.make_async_copy(
                bufs[1 - b_last], acc_sh.at[dst_v.at[nch - 2]],
                ssems[1 - b_last]).wait()
        plsc.subcore_barrier()

        for k in range(rpt // lane):
            r0 = s * rpt + k * lane
            pltpu.sync_copy(acc_sh.at[pl.ds(r0, lane)], buf0_v)
            pltpu.sync_copy(buf0_v, out_hbm.at[c, pl.ds(r0, lane)])

    return edge_kernel(y, dis, src3, dst3, ew3)


def _tc1_body(x_ref, w_ref, dp_ref, ya_ref, yb_ref, dis_ref):
    xw = jnp.dot(x_ref[...], w_ref[...], preferred_element_type=jnp.float32)
    half = xw.shape[1] // 2
    ya_ref[...] = xw[:, :half]
    yb_ref[...] = xw[:, half:]
    deg = jnp.sum(dp_ref[...], axis=0)
    dis_ref[...] = jnp.where(deg > 0, lax.rsqrt(deg), 0.0)


def _tc_matmul_dis(x_p, w1, deg_part, br):
    n_pad, d = x_p.shape
    h = w1.shape[1]
    nw = deg_part.shape[0]
    return pl.pallas_call(
        _tc1_body,
        grid=(n_pad // br,),
        in_specs=[
            pl.BlockSpec((br, d), lambda i: (i, 0)),
            pl.BlockSpec((d, h), lambda i: (0, 0)),
            pl.BlockSpec((nw, br), lambda i: (0, i)),
        ],
        out_specs=[
            pl.BlockSpec((br, h // 2), lambda i: (i, 0)),
            pl.BlockSpec((br, h // 2), lambda i: (i, 0)),
            pl.BlockSpec((br,), lambda i: (i,)),
        ],
        out_shape=[
            jax.ShapeDtypeStruct((n_pad, h // 2), jnp.float32),
            jax.ShapeDtypeStruct((n_pad, h // 2), jnp.float32),
            jax.ShapeDtypeStruct((n_pad,), jnp.float32),
        ],
    )(x_p, w1, deg_part)


def _tc2_body(aa_ref, ab_ref, b1_ref, w2_ref, y2_ref):
    hid = jnp.concatenate([aa_ref[0] + aa_ref[1], ab_ref[0] + ab_ref[1]],
                          axis=1)
    hid = jnp.maximum(hid + b1_ref[...], 0.0)
    y2_ref[...] = jnp.dot(hid, w2_ref[...], preferred_element_type=jnp.float32)


def _tc_hidden(acc1a, acc1b, b1, w2p, br):
    _, n_pad, hh = acc1a.shape
    op = w2p.shape[1]
    return pl.pallas_call(
        _tc2_body,
        grid=(n_pad // br,),
        in_specs=[
            pl.BlockSpec((2, br, hh), lambda i: (0, i, 0)),
            pl.BlockSpec((2, br, hh), lambda i: (0, i, 0)),
            pl.BlockSpec((2 * hh,), lambda i: (0,)),
            pl.BlockSpec((2 * hh, op), lambda i: (0, 0)),
        ],
        out_specs=pl.BlockSpec((br, op), lambda i: (i, 0)),
        out_shape=jax.ShapeDtypeStruct((n_pad, op), jnp.float32),
    )(acc1a, acc1b, b1, w2p)


def _tc3_body(a_ref, b2_ref, o_ref):
    o_ref[...] = a_ref[0] + a_ref[1] + b2_ref[...]


def _tc_final(acc2, b2p, br):
    _, n_pad, op = acc2.shape
    return pl.pallas_call(
        _tc3_body,
        grid=(n_pad // br,),
        in_specs=[
            pl.BlockSpec((2, br, op), lambda i: (0, i, 0)),
            pl.BlockSpec((op,), lambda i: (0,)),
        ],
        out_specs=pl.BlockSpec((br, op), lambda i: (i, 0)),
        out_shape=jax.ShapeDtypeStruct((n_pad, op), jnp.float32),
    )(acc2, b2p)


def kernel(x, edge_index, edge_weight, W1, b1, W2, b2):
    n, d = x.shape
    o = W2.shape[1]
    br = 1024
    n_pad = -(-n // br) * br
    o_p = -(-o // VREG) * VREG
    e = edge_weight.shape[0]
    e_tot = e + n_pad
    ept = -(-e_tot // (NW * LANE)) * LANE
    e_pad = ept * NW
    nch = ept // LANE

    src = edge_index[0].astype(jnp.int32)
    dst = edge_index[1].astype(jnp.int32)
    loop_idx = jnp.arange(n_pad, dtype=jnp.int32)
    zpad_i = jnp.zeros((e_pad - e_tot,), jnp.int32)
    src3 = jnp.concatenate([src, loop_idx, zpad_i]).reshape(NW, nch, LANE)
    dst3 = jnp.concatenate([dst, loop_idx, zpad_i]).reshape(NW, nch, LANE)
    ew3 = jnp.concatenate([
        edge_weight.astype(jnp.float32),
        jnp.ones((n_pad,), jnp.float32),
        jnp.zeros((e_pad - e_tot,), jnp.float32),
    ]).reshape(NW, nch, LANE)
    x_p = jnp.pad(x, ((0, n_pad - n), (0, 0)))
    w2p = jnp.pad(W2, ((0, 0), (0, o_p - o)))
    b2p = jnp.pad(b2, ((0, o_p - o),))

    deg_part = _sc_degree(dst3, ew3, n_pad)
    y1a, y1b, dis = _tc_matmul_dis(x_p, W1, deg_part, br)
    acc1a = _sc_edge_pass(y1a, dis, src3, dst3, ew3)
    acc1b = _sc_edge_pass(y1b, dis, src3, dst3, ew3)
    y2 = _tc_hidden(acc1a, acc1b, b1, w2p, br)
    acc2 = _sc_edge_pass(y2, dis, src3, dst3, ew3)
    outp = _tc_final(acc2, b2p, br)
    return outp[:n, :o]


# speedup vs baseline: 5.3232x; 1.1611x over previous
import functools

import numpy as np

import jax
import jax.numpy as jnp
from jax import lax
from jax.experimental import pallas as pl
from jax.experimental.pallas import tpu as pltpu
from jax.experimental.pallas import tpu_sc as plsc

LANE = 128
NC = 2
NS = 16
NW = NC * NS
VREG = 16
DW = 64

_PERM32 = np.concatenate([np.arange(0, 32, 2), np.arange(1, 32, 2)])
_PERM64 = np.concatenate([_PERM32, _PERM32 + 32])
_INV64 = np.argsort(_PERM64)


def _sc_degree(dst3, ew3, n_pad):
    nw, nch, lane = dst3.shape
    rpt = n_pad // NS
    mesh = plsc.VectorSubcoreMesh(core_axis_name="c", subcore_axis_name="s")

    @functools.partial(
        pl.kernel,
        out_type=jax.ShapeDtypeStruct((NC, n_pad), jnp.float32),
        mesh=mesh,
        scratch_types=[
            pltpu.VMEM((nch, lane), jnp.int32),
            pltpu.VMEM((nch, lane), jnp.float32),
            pltpu.VMEM((rpt,), jnp.float32),
            pltpu.VMEM_SHARED((n_pad,), jnp.float32),
        ],
        compiler_params=pltpu.CompilerParams(
            needs_layout_passes=False, use_tc_tiling_on_sc=False),
    )
    def deg_kernel(dst_hbm, ew_hbm, out_hbm, dst_v, ew_v, buf_v, acc_sh):
        c = lax.axis_index("c")
        s = lax.axis_index("s")
        w = s * NC + c
        pltpu.sync_copy(dst_hbm.at[w], dst_v)
        pltpu.sync_copy(ew_hbm.at[w], ew_v)

        def zero_body(i, carry):
            buf_v[pl.ds(i * VREG, VREG)] = jnp.zeros((VREG,), jnp.float32)
            return carry

        lax.fori_loop(0, rpt // VREG, zero_body, 0)
        pltpu.sync_copy(buf_v, acc_sh.at[pl.ds(s * rpt, rpt)])
        plsc.subcore_barrier()

        def edge_body(j, carry):
            pltpu.sync_copy(ew_v.at[j], acc_sh.at[dst_v.at[j]], add=True)
            return carry

        lax.fori_loop(0, nch, edge_body, 0)
        plsc.subcore_barrier()

        pltpu.sync_copy(acc_sh.at[pl.ds(s * rpt, rpt)], buf_v)
        pltpu.sync_copy(buf_v, out_hbm.at[c, pl.ds(s * rpt, rpt)])

    return deg_kernel(dst3, ew3)


def _sc_edge_pass(y_bf, dis, src3, dst3, ew3):
    n_pad, d = y_bf.shape
    nw, nch, lane = src3.shape
    rpt = n_pad // NS
    mesh = plsc.VectorSubcoreMesh(core_axis_name="c", subcore_axis_name="s")

    @functools.partial(
        pl.kernel,
        out_type=jax.ShapeDtypeStruct((NC, n_pad, d), jnp.float32),
        mesh=mesh,
        scratch_types=[
            pltpu.VMEM((nch, lane), jnp.int32),
            pltpu.VMEM((nch, lane), jnp.int32),
            pltpu.VMEM((nch, lane), jnp.float32),
            pltpu.VMEM((n_pad,), jnp.float32),
            pltpu.VMEM((lane, d), jnp.bfloat16),
            pltpu.VMEM((lane, d), jnp.bfloat16),
            pltpu.VMEM((lane, d), jnp.float32),
            pltpu.VMEM((lane, d), jnp.float32),
            pltpu.VMEM_SHARED((n_pad, d), jnp.float32),
            pltpu.SemaphoreType.DMA,
            pltpu.SemaphoreType.DMA,
            pltpu.SemaphoreType.DMA,
            pltpu.SemaphoreType.DMA,
        ],
        compiler_params=pltpu.CompilerParams(
            needs_layout_passes=False, use_tc_tiling_on_sc=False),
    )
    def edge_kernel(y_hbm, dis_hbm, src_hbm, dst_hbm, ew_hbm, out_hbm,
                    src_v, dst_v, ew_v, dis_v, gbuf0, gbuf1, sbuf0, sbuf1,
                    acc_sh, gsem0, gsem1, ssem0, ssem1):
        c = lax.axis_index("c")
        s = lax.axis_index("s")
        w = s * NC + c
        gbufs = (gbuf0, gbuf1)
        sbufs = (sbuf0, sbuf1)
        gsems = (gsem0, gsem1)
        ssems = (ssem0, ssem1)
        pltpu.sync_copy(src_hbm.at[w], src_v)
        pltpu.sync_copy(dst_hbm.at[w], dst_v)
        pltpu.sync_copy(ew_hbm.at[w], ew_v)
        pltpu.sync_copy(dis_hbm, dis_v)

        def zrow(i, carry):
            for v in range(d // VREG):
                sbuf0[i, pl.ds(v * VREG, VREG)] = jnp.zeros((VREG,), jnp.float32)
            return carry

        lax.fori_loop(0, lane, zrow, 0)
        for k in range(rpt // lane):
            pltpu.sync_copy(sbuf0, acc_sh.at[pl.ds(s * rpt + k * lane, lane)])
        plsc.subcore_barrier()

        def scale_rows(j, gbuf, sbuf):
            for g in range(lane // VREG):
                sl = pl.ds(g * VREG, VREG)
                nsrc = plsc.load_gather(dis_v, [src_v[j, sl]])
                ndst = plsc.load_gather(dis_v, [dst_v[j, sl]])
                nv = nsrc * ew_v[j, sl] * ndst
                for l in range(VREG):
                    e = g * VREG + l
                    scale = nv[l]
                    for k in range(d // 32):
                        x32 = gbuf[e, pl.ds(32 * k, 32)]
                        a, b = plsc.unpack(
                            x32, format=plsc.PackFormat.INTERLEAVED)
                        sbuf[e, pl.ds(32 * k, VREG)] = a * scale
                        sbuf[e, pl.ds(32 * k + VREG, VREG)] = b * scale

        pltpu.async_copy(y_hbm.at[src_v.at[0]], gbuf0, gsem0)

        def chunk_body(j, carry):
            def run(bb):
                @pl.when(j + 1 < nch)
                def _():
                    pltpu.async_copy(
                        y_hbm.at[src_v.at[j + 1]], gbufs[1 - bb],
                        gsems[1 - bb])

                pltpu.make_async_copy(
                    y_hbm.at[src_v.at[j]], gbufs[bb], gsems[bb]).wait()

                @pl.when(j >= 2)
                def _():
                    pltpu.make_async_copy(
                        sbufs[bb], acc_sh.at[dst_v.at[j - 2]],
                        ssems[bb]).wait()

                scale_rows(j, gbufs[bb], sbufs[bb])
                pltpu.async_copy(
                    sbufs[bb], acc_sh.at[dst_v.at[j]], ssems[bb], add=True)

            @pl.when(j % 2 == 0)
            def _():
                run(0)

            @pl.when(j % 2 == 1)
            def _():
                run(1)

            return carry

        lax.fori_loop(0, nch, chunk_body, 0)
        b_last = (nch - 1) % 2
        pltpu.make_async_copy(
            sbufs[b_last], acc_sh.at[dst_v.at[nch - 1]], ssems[b_last]).wait()
        if nch >= 2:
            pltpu.make_async_copy(
                sbufs[1 - b_last], acc_sh.at[dst_v.at[nch - 2]],
                ssems[1 - b_last]).wait()
        plsc.subcore_barrier()

        for k in range(rpt // lane):
            r0 = s * rpt + k * lane
            pltpu.sync_copy(acc_sh.at[pl.ds(r0, lane)], sbuf0)
            pltpu.sync_copy(sbuf0, out_hbm.at[c, pl.ds(r0, lane)])

    return edge_kernel(y_bf, dis, src3, dst3, ew3)


def _tc1_body(x_ref, w_ref, dp_ref, ya_ref, yb_ref, dis_ref):
    xw = jnp.dot(x_ref[...], w_ref[...], preferred_element_type=jnp.float32)
    ya_ref[...] = xw[:, :DW].astype(jnp.bfloat16)
    yb_ref[...] = xw[:, DW:].astype(jnp.bfloat16)
    deg = jnp.sum(dp_ref[...], axis=0)
    dis_ref[...] = jnp.where(deg > 0, lax.rsqrt(deg), 0.0)


def _tc_matmul_dis(x_p, w1, deg_part, br):
    n_pad, d = x_p.shape
    h = w1.shape[1]
    nw = deg_part.shape[0]
    return pl.pallas_call(
        _tc1_body,
        grid=(n_pad // br,),
        in_specs=[
            pl.BlockSpec((br, d), lambda i: (i, 0)),
            pl.BlockSpec((d, h), lambda i: (0, 0)),
            pl.BlockSpec((nw, br), lambda i: (0, i)),
        ],
        out_specs=[
            pl.BlockSpec((br, DW), lambda i: (i, 0)),
            pl.BlockSpec((br, DW), lambda i: (i, 0)),
            pl.BlockSpec((br,), lambda i: (i,)),
        ],
        out_shape=[
            jax.ShapeDtypeStruct((n_pad, DW), jnp.bfloat16),
            jax.ShapeDtypeStruct((n_pad, DW), jnp.bfloat16),
            jax.ShapeDtypeStruct((n_pad,), jnp.float32),
        ],
    )(x_p, w1, deg_part)


def _tc2_body(aa_ref, ab_ref, b1_ref, w2_ref, y2_ref):
    hid = jnp.concatenate([aa_ref[0] + aa_ref[1], ab_ref[0] + ab_ref[1]],
                          axis=1)
    hid = jnp.maximum(hid + b1_ref[...], 0.0)
    y2 = jnp.dot(hid, w2_ref[...], preferred_element_type=jnp.float32)
    y2_ref[...] = y2.astype(jnp.bfloat16)


def _tc_hidden(acc1a, acc1b, b1, w2p, br):
    _, n_pad, hh = acc1a.shape
    op = w2p.shape[1]
    return pl.pallas_call(
        _tc2_body,
        grid=(n_pad // br,),
        in_specs=[
            pl.BlockSpec((2, br, hh), lambda i: (0, i, 0)),
            pl.BlockSpec((2, br, hh), lambda i: (0, i, 0)),
            pl.BlockSpec((2 * hh,), lambda i: (0,)),
            pl.BlockSpec((2 * hh, op), lambda i: (0, 0)),
        ],
        out_specs=pl.BlockSpec((br, op), lambda i: (i, 0)),
        out_shape=jax.ShapeDtypeStruct((n_pad, op), jnp.bfloat16),
    )(acc1a, acc1b, b1, w2p)


def _tc3_body(a_ref, b2_ref, o_ref):
    o_ref[...] = a_ref[0] + a_ref[1] + b2_ref[...]


def _tc_final(acc2, b2p, br):
    _, n_pad, op = acc2.shape
    return pl.pallas_call(
        _tc3_body,
        grid=(n_pad // br,),
        in_specs=[
            pl.BlockSpec((2, br, op), lambda i: (0, i, 0)),
            pl.BlockSpec((op,), lambda i: (0,)),
        ],
        out_specs=pl.BlockSpec((br, op), lambda i: (i, 0)),
        out_shape=jax.ShapeDtypeStruct((n_pad, op), jnp.float32),
    )(acc2, b2p)


def kernel(x, edge_index, edge_weight, W1, b1, W2, b2):
    n, d = x.shape
    o = W2.shape[1]
    br = 1024
    n_pad = -(-n // br) * br
    e = edge_weight.shape[0]
    e_tot = e + n_pad
    ept = -(-e_tot // (NW * LANE)) * LANE
    e_pad = ept * NW
    nch = ept // LANE

    src = edge_index[0].astype(jnp.int32)
    dst = edge_index[1].astype(jnp.int32)
    loop_idx = jnp.arange(n_pad, dtype=jnp.int32)
    zpad_i = jnp.zeros((e_pad - e_tot,), jnp.int32)
    src3 = jnp.concatenate([src, loop_idx, zpad_i]).reshape(NW, nch, LANE)
    dst3 = jnp.concatenate([dst, loop_idx, zpad_i]).reshape(NW, nch, LANE)
    ew3 = jnp.concatenate([
        edge_weight.astype(jnp.float32),
        jnp.ones((n_pad,), jnp.float32),
        jnp.zeros((e_pad - e_tot,), jnp.float32),
    ]).reshape(NW, nch, LANE)
    x_p = jnp.pad(x, ((0, n_pad - n), (0, 0)))

    w1p = W1[:, np.concatenate([_INV64, _INV64 + DW])]
    w2p = jnp.pad(W2, ((0, 0), (0, DW - o)))[:, _INV64]
    b2p = jnp.pad(b2, ((0, DW - o),))

    deg_part = _sc_degree(dst3, ew3, n_pad)
    y1a, y1b, dis = _tc_matmul_dis(x_p, w1p, deg_part, br)
    acc1a = _sc_edge_pass(y1a, dis, src3, dst3, ew3)
    acc1b = _sc_edge_pass(y1b, dis, src3, dst3, ew3)
    y2 = _tc_hidden(acc1a, acc1b, b1, w2p, br)
    acc2 = _sc_edge_pass(y2, dis, src3, dst3, ew3)
    outp = _tc_final(acc2, b2p, br)
    return outp[:n, :o]


# speedup vs baseline: 5.5314x; 1.0391x over previous
import functools

import numpy as np

import jax
import jax.numpy as jnp
from jax import lax
from jax.experimental import pallas as pl
from jax.experimental.pallas import tpu as pltpu
from jax.experimental.pallas import tpu_sc as plsc

LANE = 128
NC = 2
NS = 16
NW = NC * NS
VREG = 16
DW = 64

_PERM32 = np.concatenate([np.arange(0, 32, 2), np.arange(1, 32, 2)])
_PERM64 = np.concatenate([_PERM32, _PERM32 + 32])
_INV64 = np.argsort(_PERM64)


def _sc_degree(dst3, ew3, n_pad):
    nw, nch, lane = dst3.shape
    rpt = n_pad // NS
    mesh = plsc.VectorSubcoreMesh(core_axis_name="c", subcore_axis_name="s")

    @functools.partial(
        pl.kernel,
        out_type=jax.ShapeDtypeStruct((NC, n_pad), jnp.float32),
        mesh=mesh,
        scratch_types=[
            pltpu.VMEM((nch, lane), jnp.int32),
            pltpu.VMEM((nch, lane), jnp.float32),
            pltpu.VMEM((rpt,), jnp.float32),
            pltpu.VMEM_SHARED((n_pad,), jnp.float32),
        ],
        compiler_params=pltpu.CompilerParams(
            needs_layout_passes=False, use_tc_tiling_on_sc=False),
    )
    def deg_kernel(dst_hbm, ew_hbm, out_hbm, dst_v, ew_v, buf_v, acc_sh):
        c = lax.axis_index("c")
        s = lax.axis_index("s")
        w = s * NC + c
        pltpu.sync_copy(dst_hbm.at[w], dst_v)
        pltpu.sync_copy(ew_hbm.at[w], ew_v)

        def zero_body(i, carry):
            buf_v[pl.ds(i * VREG, VREG)] = jnp.zeros((VREG,), jnp.float32)
            return carry

        lax.fori_loop(0, rpt // VREG, zero_body, 0)
        pltpu.sync_copy(buf_v, acc_sh.at[pl.ds(s * rpt, rpt)])
        plsc.subcore_barrier()

        def edge_body(j, carry):
            pltpu.sync_copy(ew_v.at[j], acc_sh.at[dst_v.at[j]], add=True)
            return carry

        lax.fori_loop(0, nch, edge_body, 0)
        plsc.subcore_barrier()

        pltpu.sync_copy(acc_sh.at[pl.ds(s * rpt, rpt)], buf_v)
        pltpu.sync_copy(buf_v, out_hbm.at[c, pl.ds(s * rpt, rpt)])

    return deg_kernel(dst3, ew3)


def _rsqrt16(x):
    i = plsc.bitcast(x, jnp.int32)
    i = jnp.int32(0x5F3759DF) - lax.shift_right_arithmetic(i, 1)
    y = plsc.bitcast(i, jnp.float32)
    for _ in range(3):
        y = y * (1.5 - 0.5 * x * y * y)
    return y


def _sc_edge_pass(y_bf, deg_part, src3, dst3, ew3):
    n_pad, d = y_bf.shape
    nw, nch, lane = src3.shape
    rpt = n_pad // NS
    mesh = plsc.VectorSubcoreMesh(core_axis_name="c", subcore_axis_name="s")

    @functools.partial(
        pl.kernel,
        out_type=jax.ShapeDtypeStruct((NC, n_pad, d), jnp.float32),
        mesh=mesh,
        scratch_types=[
            pltpu.VMEM((nch, lane), jnp.int32),
            pltpu.VMEM((nch, lane), jnp.int32),
            pltpu.VMEM((nch, lane), jnp.float32),
            pltpu.VMEM((n_pad,), jnp.float32),
            pltpu.VMEM((rpt,), jnp.float32),
            pltpu.VMEM((rpt,), jnp.float32),
            pltpu.VMEM((lane, d), jnp.bfloat16),
            pltpu.VMEM((lane, d), jnp.bfloat16),
            pltpu.VMEM((lane, d), jnp.float32),
            pltpu.VMEM((lane, d), jnp.float32),
            pltpu.VMEM_SHARED((n_pad,), jnp.float32),
            pltpu.VMEM_SHARED((n_pad, d), jnp.float32),
            pltpu.SemaphoreType.DMA,
            pltpu.SemaphoreType.DMA,
            pltpu.SemaphoreType.DMA,
            pltpu.SemaphoreType.DMA,
        ],
        compiler_params=pltpu.CompilerParams(
            needs_layout_passes=False, use_tc_tiling_on_sc=False),
    )
    def edge_kernel(y_hbm, degp_hbm, src_hbm, dst_hbm, ew_hbm, out_hbm,
                    src_v, dst_v, ew_v, dis_v, da_v, db_v,
                    gbuf0, gbuf1, sbuf0, sbuf1, dis_sh,
                    acc_sh, gsem0, gsem1, ssem0, ssem1):
        c = lax.axis_index("c")
        s = lax.axis_index("s")
        w = s * NC + c
        gbufs = (gbuf0, gbuf1)
        sbufs = (sbuf0, sbuf1)
        gsems = (gsem0, gsem1)
        ssems = (ssem0, ssem1)
        pltpu.sync_copy(src_hbm.at[w], src_v)
        pltpu.sync_copy(dst_hbm.at[w], dst_v)
        pltpu.sync_copy(ew_hbm.at[w], ew_v)

        pltpu.sync_copy(degp_hbm.at[0, pl.ds(s * rpt, rpt)], da_v)
        pltpu.sync_copy(degp_hbm.at[1, pl.ds(s * rpt, rpt)], db_v)

        def dis_body(i, carry):
            sl = pl.ds(i * VREG, VREG)
            da_v[sl] = _rsqrt16(da_v[sl] + db_v[sl])
            return carry

        lax.fori_loop(0, rpt // VREG, dis_body, 0)
        pltpu.sync_copy(da_v, dis_sh.at[pl.ds(s * rpt, rpt)])

        def zrow(i, carry):
            for v in range(d // VREG):
                sbuf0[i, pl.ds(v * VREG, VREG)] = jnp.zeros((VREG,), jnp.float32)
            return carry

        lax.fori_loop(0, lane, zrow, 0)
        for k in range(rpt // lane):
            pltpu.sync_copy(sbuf0, acc_sh.at[pl.ds(s * rpt + k * lane, lane)])
        plsc.subcore_barrier()
        pltpu.sync_copy(dis_sh, dis_v)

        def scale_rows(j, gbuf, sbuf):
            for g in range(lane // VREG):
                sl = pl.ds(g * VREG, VREG)
                nsrc = plsc.load_gather(dis_v, [src_v[j, sl]])
                ndst = plsc.load_gather(dis_v, [dst_v[j, sl]])
                nv = nsrc * ew_v[j, sl] * ndst
                for l in range(VREG):
                    e = g * VREG + l
                    scale = nv[l]
                    for k in range(d // 32):
                        x32 = gbuf[e, pl.ds(32 * k, 32)]
                        a, b = plsc.unpack(
                            x32, format=plsc.PackFormat.INTERLEAVED)
                        sbuf[e, pl.ds(32 * k, VREG)] = a * scale
                        sbuf[e, pl.ds(32 * k + VREG, VREG)] = b * scale

        pltpu.async_copy(y_hbm.at[src_v.at[0]], gbuf0, gsem0)

        def chunk_body(j, carry):
            def run(bb):
                @pl.when(j + 1 < nch)
                def _():
                    pltpu.async_copy(
                        y_hbm.at[src_v.at[j + 1]], gbufs[1 - bb],
                        gsems[1 - bb])

                pltpu.make_async_copy(
                    y_hbm.at[src_v.at[j]], gbufs[bb], gsems[bb]).wait()

                @pl.when(j >= 2)
                def _():
                    pltpu.make_async_copy(
                        sbufs[bb], acc_sh.at[dst_v.at[j - 2]],
                        ssems[bb]).wait()

                scale_rows(j, gbufs[bb], sbufs[bb])
                pltpu.async_copy(
                    sbufs[bb], acc_sh.at[dst_v.at[j]], ssems[bb], add=True)

            @pl.when(j % 2 == 0)
            def _():
                run(0)

            @pl.when(j % 2 == 1)
            def _():
                run(1)

            return carry

        lax.fori_loop(0, nch, chunk_body, 0)
        b_last = (nch - 1) % 2
        pltpu.make_async_copy(
            sbufs[b_last], acc_sh.at[dst_v.at[nch - 1]], ssems[b_last]).wait()
        if nch >= 2:
            pltpu.make_async_copy(
                sbufs[1 - b_last], acc_sh.at[dst_v.at[nch - 2]],
                ssems[1 - b_last]).wait()
        plsc.subcore_barrier()

        for k in range(rpt // lane):
            r0 = s * rpt + k * lane
            pltpu.sync_copy(acc_sh.at[pl.ds(r0, lane)], sbuf0)
            pltpu.sync_copy(sbuf0, out_hbm.at[c, pl.ds(r0, lane)])

    return edge_kernel(y_bf, deg_part, src3, dst3, ew3)


def _tc1_body(x_ref, w_ref, ya_ref, yb_ref):
    xw = jnp.dot(x_ref[...], w_ref[...], preferred_element_type=jnp.float32)
    ya_ref[...] = xw[:, :DW].astype(jnp.bfloat16)
    yb_ref[...] = xw[:, DW:].astype(jnp.bfloat16)


def _tc_matmul(x_p, w1, br):
    n_pad, d = x_p.shape
    h = w1.shape[1]
    return pl.pallas_call(
        _tc1_body,
        grid=(n_pad // br,),
        in_specs=[
            pl.BlockSpec((br, d), lambda i: (i, 0)),
            pl.BlockSpec((d, h), lambda i: (0, 0)),
        ],
        out_specs=[
            pl.BlockSpec((br, DW), lambda i: (i, 0)),
            pl.BlockSpec((br, DW), lambda i: (i, 0)),
        ],
        out_shape=[
            jax.ShapeDtypeStruct((n_pad, DW), jnp.bfloat16),
            jax.ShapeDtypeStruct((n_pad, DW), jnp.bfloat16),
        ],
    )(x_p, w1)


def _tc2_body(aa_ref, ab_ref, b1_ref, w2_ref, y2_ref):
    hid = jnp.concatenate([aa_ref[0] + aa_ref[1], ab_ref[0] + ab_ref[1]],
                          axis=1)
    hid = jnp.maximum(hid + b1_ref[...], 0.0)
    y2 = jnp.dot(hid, w2_ref[...], preferred_element_type=jnp.float32)
    y2_ref[...] = y2.astype(jnp.bfloat16)


def _tc_hidden(acc1a, acc1b, b1, w2p, br):
    _, n_pad, hh = acc1a.shape
    op = w2p.shape[1]
    return pl.pallas_call(
        _tc2_body,
        grid=(n_pad // br,),
        in_specs=[
            pl.BlockSpec((2, br, hh), lambda i: (0, i, 0)),
            pl.BlockSpec((2, br, hh), lambda i: (0, i, 0)),
            pl.BlockSpec((2 * hh,), lambda i: (0,)),
            pl.BlockSpec((2 * hh, op), lambda i: (0, 0)),
        ],
        out_specs=pl.BlockSpec((br, op), lambda i: (i, 0)),
        out_shape=jax.ShapeDtypeStruct((n_pad, op), jnp.bfloat16),
    )(acc1a, acc1b, b1, w2p)


def _tc3_body(a_ref, b2_ref, o_ref):
    o_ref[...] = a_ref[0] + a_ref[1] + b2_ref[...]


def _tc_final(acc2, b2p, br):
    _, n_pad, op = acc2.shape
    return pl.pallas_call(
        _tc3_body,
        grid=(n_pad // br,),
        in_specs=[
            pl.BlockSpec((2, br, op), lambda i: (0, i, 0)),
            pl.BlockSpec((op,), lambda i: (0,)),
        ],
        out_specs=pl.BlockSpec((br, op), lambda i: (i, 0)),
        out_shape=jax.ShapeDtypeStruct((n_pad, op), jnp.float32),
    )(acc2, b2p)


def kernel(x, edge_index, edge_weight, W1, b1, W2, b2):
    n, d = x.shape
    o = W2.shape[1]
    br = 1024
    n_pad = -(-n // br) * br
    e = edge_weight.shape[0]
    e_tot = e + n_pad
    ept = -(-e_tot // (NW * LANE)) * LANE
    e_pad = ept * NW
    nch = ept // LANE

    src = edge_index[0].astype(jnp.int32)
    dst = edge_index[1].astype(jnp.int32)
    loop_idx = jnp.arange(n_pad, dtype=jnp.int32)
    zpad_i = jnp.zeros((e_pad - e_tot,), jnp.int32)
    src3 = jnp.concatenate([src, loop_idx, zpad_i]).reshape(NW, nch, LANE)
    dst3 = jnp.concatenate([dst, loop_idx, zpad_i]).reshape(NW, nch, LANE)
    ew3 = jnp.concatenate([
        edge_weight.astype(jnp.float32),
        jnp.ones((n_pad,), jnp.float32),
        jnp.zeros((e_pad - e_tot,), jnp.float32),
    ]).reshape(NW, nch, LANE)
    x_p = jnp.pad(x, ((0, n_pad - n), (0, 0)))

    w1p = W1[:, np.concatenate([_INV64, _INV64 + DW])]
    w2p = jnp.pad(W2, ((0, 0), (0, DW - o)))[:, _INV64]
    b2p = jnp.pad(b2, ((0, DW - o),))

    deg_part = _sc_degree(dst3, ew3, n_pad)
    y1a, y1b = _tc_matmul(x_p, w1p, br)
    acc1a = _sc_edge_pass(y1a, deg_part, src3, dst3, ew3)
    acc1b = _sc_edge_pass(y1b, deg_part, src3, dst3, ew3)
    y2 = _tc_hidden(acc1a, acc1b, b1, w2p, br)
    acc2 = _sc_edge_pass(y2, deg_part, src3, dst3, ew3)
    outp = _tc_final(acc2, b2p, br)
    return outp[:n, :o]


# speedup vs baseline: 5.7732x; 1.0437x over previous
import functools

import numpy as np

import jax
import jax.numpy as jnp
from jax import lax
from jax.experimental import pallas as pl
from jax.experimental.pallas import tpu as pltpu
from jax.experimental.pallas import tpu_sc as plsc

LANE = 128
NC = 2
NS = 16
NW = NC * NS
VREG = 16
DW = 64

_PERM32 = np.concatenate([np.arange(0, 32, 2), np.arange(1, 32, 2)])
_PERM64 = np.concatenate([_PERM32, _PERM32 + 32])
_INV64 = np.argsort(_PERM64)


def _sc_degree(dst3, ew3, n_pad):
    nw, nch, lane = dst3.shape
    rpt = n_pad // NS
    mesh = plsc.VectorSubcoreMesh(core_axis_name="c", subcore_axis_name="s")

    @functools.partial(
        pl.kernel,
        out_type=jax.ShapeDtypeStruct((NC, n_pad), jnp.float32),
        mesh=mesh,
        scratch_types=[
            pltpu.VMEM((nch, lane), jnp.int32),
            pltpu.VMEM((nch, lane), jnp.float32),
            pltpu.VMEM((rpt,), jnp.float32),
            pltpu.VMEM_SHARED((n_pad,), jnp.float32),
        ],
        compiler_params=pltpu.CompilerParams(
            needs_layout_passes=False, use_tc_tiling_on_sc=False),
    )
    def deg_kernel(dst_hbm, ew_hbm, out_hbm, dst_v, ew_v, buf_v, acc_sh):
        c = lax.axis_index("c")
        s = lax.axis_index("s")
        w = s * NC + c
        pltpu.sync_copy(dst_hbm.at[w], dst_v)
        pltpu.sync_copy(ew_hbm.at[w], ew_v)

        def zero_body(i, carry):
            buf_v[pl.ds(i * VREG, VREG)] = jnp.zeros((VREG,), jnp.float32)
            return carry

        lax.fori_loop(0, rpt // VREG, zero_body, 0)
        pltpu.sync_copy(buf_v, acc_sh.at[pl.ds(s * rpt, rpt)])
        plsc.subcore_barrier()

        def edge_body(j, carry):
            pltpu.sync_copy(ew_v.at[j], acc_sh.at[dst_v.at[j]], add=True)
            return carry

        lax.fori_loop(0, nch, edge_body, 0)
        plsc.subcore_barrier()

        pltpu.sync_copy(acc_sh.at[pl.ds(s * rpt, rpt)], buf_v)
        pltpu.sync_copy(buf_v, out_hbm.at[c, pl.ds(s * rpt, rpt)])

    return deg_kernel(dst3, ew3)


def _rsqrt16(x):
    i = plsc.bitcast(x, jnp.int32)
    i = jnp.int32(0x5F3759DF) - lax.shift_right_arithmetic(i, 1)
    y = plsc.bitcast(i, jnp.float32)
    for _ in range(3):
        y = y * (1.5 - 0.5 * x * y * y)
    return y


def _sc_edge_pass(y_bf, deg_part, src3, dst3, ew3):
    n_pad, d = y_bf.shape
    nw, nch, lane = src3.shape
    rpt = n_pad // NS
    mesh = plsc.VectorSubcoreMesh(core_axis_name="c", subcore_axis_name="s")

    @functools.partial(
        pl.kernel,
        out_type=jax.ShapeDtypeStruct((NC, n_pad, d), jnp.float32),
        mesh=mesh,
        scratch_types=[
            pltpu.VMEM((nch, lane), jnp.int32),
            pltpu.VMEM((nch, lane), jnp.int32),
            pltpu.VMEM((nch, lane), jnp.float32),
            pltpu.VMEM((n_pad,), jnp.float32),
            pltpu.VMEM((rpt,), jnp.float32),
            pltpu.VMEM((rpt,), jnp.float32),
            pltpu.VMEM((lane, d), jnp.bfloat16),
            pltpu.VMEM((lane, d), jnp.bfloat16),
            pltpu.VMEM((lane, d), jnp.float32),
            pltpu.VMEM((lane, d), jnp.float32),
            pltpu.VMEM_SHARED((n_pad,), jnp.float32),
            pltpu.VMEM_SHARED((n_pad, d), jnp.float32),
            pltpu.SemaphoreType.DMA,
            pltpu.SemaphoreType.DMA,
            pltpu.SemaphoreType.DMA,
            pltpu.SemaphoreType.DMA,
            pltpu.SemaphoreType.DMA,
        ],
        compiler_params=pltpu.CompilerParams(
            needs_layout_passes=False, use_tc_tiling_on_sc=False),
    )
    def edge_kernel(y_hbm, degp_hbm, src_hbm, dst_hbm, ew_hbm, out_hbm,
                    src_v, dst_v, ew_v, dis_v, da_v, db_v,
                    gbuf0, gbuf1, sbuf0, sbuf1, dis_sh,
                    acc_sh, gsem0, gsem1, ssem0, ssem1, psem):
        c = lax.axis_index("c")
        s = lax.axis_index("s")
        w = s * NC + c
        gbufs = (gbuf0, gbuf1)
        sbufs = (sbuf0, sbuf1)
        gsems = (gsem0, gsem1)
        ssems = (ssem0, ssem1)
        loads = [
            pltpu.async_copy(src_hbm.at[w], src_v, psem),
            pltpu.async_copy(dst_hbm.at[w], dst_v, psem),
            pltpu.async_copy(ew_hbm.at[w], ew_v, psem),
            pltpu.async_copy(degp_hbm.at[0, pl.ds(s * rpt, rpt)], da_v, psem),
            pltpu.async_copy(degp_hbm.at[1, pl.ds(s * rpt, rpt)], db_v, psem),
        ]
        for ld in loads:
            ld.wait()

        def dis_body(i, carry):
            sl = pl.ds(i * VREG, VREG)
            da_v[sl] = _rsqrt16(da_v[sl] + db_v[sl])
            return carry

        lax.fori_loop(0, rpt // VREG, dis_body, 0)
        pltpu.sync_copy(da_v, dis_sh.at[pl.ds(s * rpt, rpt)])

        def zrow(i, carry):
            for v in range(d // VREG):
                sbuf0[i, pl.ds(v * VREG, VREG)] = jnp.zeros((VREG,), jnp.float32)
            return carry

        lax.fori_loop(0, lane, zrow, 0)
        for k in range(rpt // lane):
            pltpu.sync_copy(sbuf0, acc_sh.at[pl.ds(s * rpt + k * lane, lane)])
        plsc.subcore_barrier()
        pltpu.sync_copy(dis_sh, dis_v)

        def scale_rows(j, gbuf, sbuf):
            for g in range(lane // VREG):
                sl = pl.ds(g * VREG, VREG)
                nsrc = plsc.load_gather(dis_v, [src_v[j, sl]])
                ndst = plsc.load_gather(dis_v, [dst_v[j, sl]])
                nv = nsrc * ew_v[j, sl] * ndst
                for l in range(VREG):
                    e = g * VREG + l
                    scale = nv[l]
                    for k in range(d // 32):
                        x32 = gbuf[e, pl.ds(32 * k, 32)]
                        a, b = plsc.unpack(
                            x32, format=plsc.PackFormat.INTERLEAVED)
                        sbuf[e, pl.ds(32 * k, VREG)] = a * scale
                        sbuf[e, pl.ds(32 * k + VREG, VREG)] = b * scale

        pltpu.async_copy(y_hbm.at[src_v.at[0]], gbuf0, gsem0)

        def chunk_body(j, carry):
            def run(bb):
                @pl.when(j + 1 < nch)
                def _():
                    pltpu.async_copy(
                        y_hbm.at[src_v.at[j + 1]], gbufs[1 - bb],
                        gsems[1 - bb])

                pltpu.make_async_copy(
                    y_hbm.at[src_v.at[j]], gbufs[bb], gsems[bb]).wait()

                @pl.when(j >= 2)
                def _():
                    pltpu.make_async_copy(
                        sbufs[bb], acc_sh.at[dst_v.at[j - 2]],
                        ssems[bb]).wait()

                scale_rows(j, gbufs[bb], sbufs[bb])
                pltpu.async_copy(
                    sbufs[bb], acc_sh.at[dst_v.at[j]], ssems[bb], add=True)

            @pl.when(j % 2 == 0)
            def _():
                run(0)

            @pl.when(j % 2 == 1)
            def _():
                run(1)

            return carry

        lax.fori_loop(0, nch, chunk_body, 0)
        b_last = (nch - 1) % 2
        pltpu.make_async_copy(
            sbufs[b_last], acc_sh.at[dst_v.at[nch - 1]], ssems[b_last]).wait()
        if nch >= 2:
            pltpu.make_async_copy(
                sbufs[1 - b_last], acc_sh.at[dst_v.at[nch - 2]],
                ssems[1 - b_last]).wait()
        plsc.subcore_barrier()

        pltpu.sync_copy(acc_sh.at[pl.ds(s * rpt, rpt)],
                        out_hbm.at[c, pl.ds(s * rpt, rpt)])

    return edge_kernel(y_bf, deg_part, src3, dst3, ew3)


def _tc1_body(x_ref, w_ref, ya_ref, yb_ref):
    xw = jnp.dot(x_ref[...], w_ref[...], preferred_element_type=jnp.float32)
    ya_ref[...] = xw[:, :DW].astype(jnp.bfloat16)
    yb_ref[...] = xw[:, DW:].astype(jnp.bfloat16)


def _tc_matmul(x_p, w1, br):
    n_pad, d = x_p.shape
    h = w1.shape[1]
    return pl.pallas_call(
        _tc1_body,
        grid=(n_pad // br,),
        in_specs=[
            pl.BlockSpec((br, d), lambda i: (i, 0)),
            pl.BlockSpec((d, h), lambda i: (0, 0)),
        ],
        out_specs=[
            pl.BlockSpec((br, DW), lambda i: (i, 0)),
            pl.BlockSpec((br, DW), lambda i: (i, 0)),
        ],
        out_shape=[
            jax.ShapeDtypeStruct((n_pad, DW), jnp.bfloat16),
            jax.ShapeDtypeStruct((n_pad, DW), jnp.bfloat16),
        ],
    )(x_p, w1)


def _tc2_body(aa_ref, ab_ref, b1_ref, w2_ref, y2_ref):
    hid = jnp.concatenate([aa_ref[0] + aa_ref[1], ab_ref[0] + ab_ref[1]],
                          axis=1)
    hid = jnp.maximum(hid + b1_ref[...], 0.0)
    y2 = jnp.dot(hid, w2_ref[...], preferred_element_type=jnp.float32)
    y2_ref[...] = y2.astype(jnp.bfloat16)


def _tc_hidden(acc1a, acc1b, b1, w2p, br):
    _, n_pad, hh = acc1a.shape
    op = w2p.shape[1]
    return pl.pallas_call(
        _tc2_body,
        grid=(n_pad // br,),
        in_specs=[
            pl.BlockSpec((2, br, hh), lambda i: (0, i, 0)),
            pl.BlockSpec((2, br, hh), lambda i: (0, i, 0)),
            pl.BlockSpec((2 * hh,), lambda i: (0,)),
            pl.BlockSpec((2 * hh, op), lambda i: (0, 0)),
        ],
        out_specs=pl.BlockSpec((br, op), lambda i: (i, 0)),
        out_shape=jax.ShapeDtypeStruct((n_pad, op), jnp.bfloat16),
    )(acc1a, acc1b, b1, w2p)


def kernel(x, edge_index, edge_weight, W1, b1, W2, b2):
    n, d = x.shape
    o = W2.shape[1]
    br = 1024
    n_pad = -(-n // br) * br
    e = edge_weight.shape[0]
    e_tot = e + n_pad
    ept = -(-e_tot // (NW * LANE)) * LANE
    e_pad = ept * NW
    nch = ept // LANE

    src = edge_index[0].astype(jnp.int32)
    dst = edge_index[1].astype(jnp.int32)
    loop_idx = jnp.arange(n_pad, dtype=jnp.int32)
    zpad_i = jnp.zeros((e_pad - e_tot,), jnp.int32)
    src3 = jnp.concatenate([src, loop_idx, zpad_i]).reshape(NW, nch, LANE)
    dst3 = jnp.concatenate([dst, loop_idx, zpad_i]).reshape(NW, nch, LANE)
    ew3 = jnp.concatenate([
        edge_weight.astype(jnp.float32),
        jnp.ones((n_pad,), jnp.float32),
        jnp.zeros((e_pad - e_tot,), jnp.float32),
    ]).reshape(NW, nch, LANE)
    x_p = jnp.pad(x, ((0, n_pad - n), (0, 0)))

    w1p = W1[:, np.concatenate([_INV64, _INV64 + DW])]
    w2p = jnp.pad(W2, ((0, 0), (0, DW - o)))[:, _INV64]

    deg_part = _sc_degree(dst3, ew3, n_pad)
    y1a, y1b = _tc_matmul(x_p, w1p, br)
    acc1a = _sc_edge_pass(y1a, deg_part, src3, dst3, ew3)
    acc1b = _sc_edge_pass(y1b, deg_part, src3, dst3, ew3)
    y2 = _tc_hidden(acc1a, acc1b, b1, w2p, br)
    acc2 = _sc_edge_pass(y2, deg_part, src3, dst3, ew3)
    return (acc2[0, :n, :o] + acc2[1, :n, :o]) + b2


# speedup vs baseline: 5.8105x; 1.0065x over previous
import functools

import numpy as np

import jax
import jax.numpy as jnp
from jax import lax
from jax.experimental import pallas as pl
from jax.experimental.pallas import tpu as pltpu
from jax.experimental.pallas import tpu_sc as plsc

LANE = 128
NC = 2
NS = 16
NW = NC * NS
VREG = 16
DW = 64

_PERM32 = np.concatenate([np.arange(0, 32, 2), np.arange(1, 32, 2)])
_PERM64 = np.concatenate([_PERM32, _PERM32 + 32])
_INV64 = np.argsort(_PERM64)


def _sc_degree(dst3, ew3, n_pad):
    nw, nch, lane = dst3.shape
    rpt = n_pad // NS
    mesh = plsc.VectorSubcoreMesh(core_axis_name="c", subcore_axis_name="s")

    @functools.partial(
        pl.kernel,
        out_type=jax.ShapeDtypeStruct((NC, n_pad), jnp.float32),
        mesh=mesh,
        scratch_types=[
            pltpu.VMEM((nch, lane), jnp.int32),
            pltpu.VMEM((nch, lane), jnp.float32),
            pltpu.VMEM((rpt,), jnp.float32),
            pltpu.VMEM_SHARED((n_pad,), jnp.float32),
        ],
        compiler_params=pltpu.CompilerParams(
            needs_layout_passes=False, use_tc_tiling_on_sc=False),
    )
    def deg_kernel(dst_hbm, ew_hbm, out_hbm, dst_v, ew_v, buf_v, acc_sh):
        c = lax.axis_index("c")
        s = lax.axis_index("s")
        w = s * NC + c
        pltpu.sync_copy(dst_hbm.at[w], dst_v)
        pltpu.sync_copy(ew_hbm.at[w], ew_v)

        def zero_body(i, carry):
            buf_v[pl.ds(i * VREG, VREG)] = jnp.zeros((VREG,), jnp.float32)
            return carry

        lax.fori_loop(0, rpt // VREG, zero_body, 0)
        pltpu.sync_copy(buf_v, acc_sh.at[pl.ds(s * rpt, rpt)])
        plsc.subcore_barrier()

        def edge_body(j, carry):
            pltpu.sync_copy(ew_v.at[j], acc_sh.at[dst_v.at[j]], add=True)
            return carry

        lax.fori_loop(0, nch, edge_body, 0)
        plsc.subcore_barrier()

        pltpu.sync_copy(acc_sh.at[pl.ds(s * rpt, rpt)], buf_v)
        pltpu.sync_copy(buf_v, out_hbm.at[c, pl.ds(s * rpt, rpt)])

    return deg_kernel(dst3, ew3)


def _rsqrt16(x):
    i = plsc.bitcast(x, jnp.int32)
    i = jnp.int32(0x5F3759DF) - lax.shift_right_arithmetic(i, 1)
    y = plsc.bitcast(i, jnp.float32)
    for _ in range(3):
        y = y * (1.5 - 0.5 * x * y * y)
    return y


def _sc_edge_pass(y_bf, deg_part, src3, dst3, ew3):
    n_pad, d = y_bf.shape
    nw, nch, lane = src3.shape
    rpt = n_pad // NS
    mesh = plsc.VectorSubcoreMesh(core_axis_name="c", subcore_axis_name="s")

    @functools.partial(
        pl.kernel,
        out_type=jax.ShapeDtypeStruct((NC, n_pad, d), jnp.float32),
        mesh=mesh,
        scratch_types=[
            pltpu.VMEM((nch, lane), jnp.int32),
            pltpu.VMEM((nch, lane), jnp.int32),
            pltpu.VMEM((nch, lane), jnp.float32),
            pltpu.VMEM((n_pad,), jnp.float32),
            pltpu.VMEM((rpt,), jnp.float32),
            pltpu.VMEM((rpt,), jnp.float32),
            pltpu.VMEM((lane, d), jnp.bfloat16),
            pltpu.VMEM((lane, d), jnp.bfloat16),
            pltpu.VMEM((lane, d), jnp.float32),
            pltpu.VMEM((lane, d), jnp.float32),
            pltpu.VMEM_SHARED((n_pad,), jnp.float32),
            pltpu.VMEM_SHARED((n_pad, d), jnp.float32),
            pltpu.SemaphoreType.DMA,
            pltpu.SemaphoreType.DMA,
            pltpu.SemaphoreType.DMA,
            pltpu.SemaphoreType.DMA,
            pltpu.SemaphoreType.DMA,
        ],
        compiler_params=pltpu.CompilerParams(
            needs_layout_passes=False, use_tc_tiling_on_sc=False),
    )
    def edge_kernel(y_hbm, degp_hbm, src_hbm, dst_hbm, ew_hbm, out_hbm,
                    src_v, dst_v, ew_v, dis_v, da_v, db_v,
                    gbuf0, gbuf1, sbuf0, sbuf1, dis_sh,
                    acc_sh, gsem0, gsem1, ssem0, ssem1, psem):
        c = lax.axis_index("c")
        s = lax.axis_index("s")
        w = s * NC + c
        gbufs = (gbuf0, gbuf1)
        sbufs = (sbuf0, sbuf1)
        gsems = (gsem0, gsem1)
        ssems = (ssem0, ssem1)
        loads = [
            pltpu.async_copy(src_hbm.at[w], src_v, psem),
            pltpu.async_copy(dst_hbm.at[w], dst_v, psem),
            pltpu.async_copy(ew_hbm.at[w], ew_v, psem),
            pltpu.async_copy(degp_hbm.at[0, pl.ds(s * rpt, rpt)], da_v, psem),
            pltpu.async_copy(degp_hbm.at[1, pl.ds(s * rpt, rpt)], db_v, psem),
        ]
        for ld in loads:
            ld.wait()
        pltpu.async_copy(y_hbm.at[src_v.at[0]], gbuf0, gsem0)

        def dis_body(i, carry):
            sl = pl.ds(i * VREG, VREG)
            da_v[sl] = _rsqrt16(da_v[sl] + db_v[sl])
            return carry

        lax.fori_loop(0, rpt // VREG, dis_body, 0)
        pltpu.sync_copy(da_v, dis_sh.at[pl.ds(s * rpt, rpt)])

        def zrow(i, carry):
            for v in range(d // VREG):
                sbuf0[i, pl.ds(v * VREG, VREG)] = jnp.zeros((VREG,), jnp.float32)
            return carry

        lax.fori_loop(0, lane, zrow, 0)
        zcopies = [
            pltpu.async_copy(
                sbuf0, acc_sh.at[pl.ds(s * rpt + k * lane, lane)], psem)
            for k in range(rpt // lane)
        ]
        for zc in zcopies:
            zc.wait()
        plsc.subcore_barrier()
        pltpu.sync_copy(dis_sh, dis_v)

        def scale_rows(j, gbuf, sbuf):
            for g in range(lane // VREG):
                sl = pl.ds(g * VREG, VREG)
                nsrc = plsc.load_gather(dis_v, [src_v[j, sl]])
                ndst = plsc.load_gather(dis_v, [dst_v[j, sl]])
                nv = nsrc * ew_v[j, sl] * ndst
                for l in range(VREG):
                    e = g * VREG + l
                    scale = nv[l]
                    for k in range(d // 32):
                        x32 = gbuf[e, pl.ds(32 * k, 32)]
                        a, b = plsc.unpack(
                            x32, format=plsc.PackFormat.INTERLEAVED)
                        sbuf[e, pl.ds(32 * k, VREG)] = a * scale
                        sbuf[e, pl.ds(32 * k + VREG, VREG)] = b * scale

        def chunk_body(j, carry):
            def run(bb):
                @pl.when(j + 1 < nch)
                def _():
                    pltpu.async_copy(
                        y_hbm.at[src_v.at[j + 1]], gbufs[1 - bb],
                        gsems[1 - bb])

                pltpu.make_async_copy(
                    y_hbm.at[src_v.at[j]], gbufs[bb], gsems[bb]).wait()

                @pl.when(j >= 2)
                def _():
                    pltpu.make_async_copy(
                        sbufs[bb], acc_sh.at[dst_v.at[j - 2]],
                        ssems[bb]).wait()

                scale_rows(j, gbufs[bb], sbufs[bb])
                pltpu.async_copy(
                    sbufs[bb], acc_sh.at[dst_v.at[j]], ssems[bb], add=True)

            @pl.when(j % 2 == 0)
            def _():
                run(0)

            @pl.when(j % 2 == 1)
            def _():
                run(1)

            return carry

        lax.fori_loop(0, nch, chunk_body, 0)
        b_last = (nch - 1) % 2
        pltpu.make_async_copy(
            sbufs[b_last], acc_sh.at[dst_v.at[nch - 1]], ssems[b_last]).wait()
        if nch >= 2:
            pltpu.make_async_copy(
                sbufs[1 - b_last], acc_sh.at[dst_v.at[nch - 2]],
                ssems[1 - b_last]).wait()
        plsc.subcore_barrier()

        pltpu.sync_copy(acc_sh.at[pl.ds(s * rpt, rpt)],
                        out_hbm.at[c, pl.ds(s * rpt, rpt)])

    return edge_kernel(y_bf, deg_part, src3, dst3, ew3)


def _tc1_body(x_ref, w_ref, ya_ref, yb_ref):
    xw = jnp.dot(x_ref[...], w_ref[...], preferred_element_type=jnp.float32)
    ya_ref[...] = xw[:, :DW].astype(jnp.bfloat16)
    yb_ref[...] = xw[:, DW:].astype(jnp.bfloat16)


def _tc_matmul(x_p, w1, br):
    n_pad, d = x_p.shape
    h = w1.shape[1]
    return pl.pallas_call(
        _tc1_body,
        grid=(n_pad // br,),
        in_specs=[
            pl.BlockSpec((br, d), lambda i: (i, 0)),
            pl.BlockSpec((d, h), lambda i: (0, 0)),
        ],
        out_specs=[
            pl.BlockSpec((br, DW), lambda i: (i, 0)),
            pl.BlockSpec((br, DW), lambda i: (i, 0)),
        ],
        out_shape=[
            jax.ShapeDtypeStruct((n_pad, DW), jnp.bfloat16),
            jax.ShapeDtypeStruct((n_pad, DW), jnp.bfloat16),
        ],
    )(x_p, w1)


def _tc2_body(aa_ref, ab_ref, b1_ref, w2_ref, y2_ref):
    hid = jnp.concatenate([aa_ref[0] + aa_ref[1], ab_ref[0] + ab_ref[1]],
                          axis=1)
    hid = jnp.maximum(hid + b1_ref[...], 0.0)
    y2 = jnp.dot(hid, w2_ref[...], preferred_element_type=jnp.float32)
    y2_ref[...] = y2.astype(jnp.bfloat16)


def _tc_hidden(acc1a, acc1b, b1, w2p, br):
    _, n_pad, hh = acc1a.shape
    op = w2p.shape[1]
    return pl.pallas_call(
        _tc2_body,
        grid=(n_pad // br,),
        in_specs=[
            pl.BlockSpec((2, br, hh), lambda i: (0, i, 0)),
            pl.BlockSpec((2, br, hh), lambda i: (0, i, 0)),
            pl.BlockSpec((2 * hh,), lambda i: (0,)),
            pl.BlockSpec((2 * hh, op), lambda i: (0, 0)),
        ],
        out_specs=pl.BlockSpec((br, op), lambda i: (i, 0)),
        out_shape=jax.ShapeDtypeStruct((n_pad, op), jnp.bfloat16),
    )(acc1a, acc1b, b1, w2p)


def kernel(x, edge_index, edge_weight, W1, b1, W2, b2):
    n, d = x.shape
    o = W2.shape[1]
    br = 1024
    n_pad = -(-n // br) * br
    e = edge_weight.shape[0]
    e_tot = e + n_pad
    ept = -(-e_tot // (NW * LANE)) * LANE
    e_pad = ept * NW
    nch = ept // LANE

    src = edge_index[0].astype(jnp.int32)
    dst = edge_index[1].astype(jnp.int32)
    loop_idx = jnp.arange(n_pad, dtype=jnp.int32)
    zpad_i = jnp.zeros((e_pad - e_tot,), jnp.int32)
    src3 = jnp.concatenate([src, loop_idx, zpad_i]).reshape(NW, nch, LANE)
    dst3 = jnp.concatenate([dst, loop_idx, zpad_i]).reshape(NW, nch, LANE)
    ew3 = jnp.concatenate([
        edge_weight.astype(jnp.float32),
        jnp.ones((n_pad,), jnp.float32),
        jnp.zeros((e_pad - e_tot,), jnp.float32),
    ]).reshape(NW, nch, LANE)
    x_p = jnp.pad(x, ((0, n_pad - n), (0, 0)))

    w1p = W1[:, np.concatenate([_INV64, _INV64 + DW])]
    w2p = jnp.pad(W2, ((0, 0), (0, DW - o)))[:, _INV64]

    deg_part = _sc_degree(dst3, ew3, n_pad)
    y1a, y1b = _tc_matmul(x_p, w1p, br)
    acc1a = _sc_edge_pass(y1a, deg_part, src3, dst3, ew3)
    acc1b = _sc_edge_pass(y1b, deg_part, src3, dst3, ew3)
    y2 = _tc_hidden(acc1a, acc1b, b1, w2p, br)
    acc2 = _sc_edge_pass(y2, deg_part, src3, dst3, ew3)
    return (acc2[0, :n, :o] + acc2[1, :n, :o]) + b2
